# baseline (speedup 1.0000x reference)
import functools

import jax
import jax.numpy as jnp
from jax import lax
from jax.experimental import pallas as pl
from jax.experimental.pallas import tpu as pltpu
from jax.experimental.pallas import tpu_sc as plsc

N = 10000
D = 128
C = 40
E = 320000

NC = 2
NS = 16
NW = NC * NS
N_PAD = 10240
RPT = N_PAD // NW
CHUNK = 64

_mesh = plsc.VectorSubcoreMesh(
    core_axis_name="c", subcore_axis_name="s", num_cores=NC, num_subcores=NS)


def _bounds_scalar(bv, t):
    total = jnp.zeros((), jnp.int32)
    for q in range(3):
        v = bv[pl.ds(q * 16, 16)]
        for lane in range(16):
            i = q * 16 + lane
            if i <= NW:
                total = jnp.where(t == i, v[lane], total)
    return total


def _make_sc_agg(width, with_gather):
    nvec = width // 16

    def body(g_hbm, src_hbm, dst_hbm, bounds_hbm, zeros_hbm, out_hbm,
             acc, sidx0, sidx1, didx0, didx1, bvm, rows, sem0, sem1):
        cid = lax.axis_index("c")
        sid = lax.axis_index("s")
        wid = cid * NS + sid
        base = wid * RPT

        pltpu.sync_copy(zeros_hbm, acc)
        pltpu.sync_copy(bounds_hbm, bvm)
        e0 = _bounds_scalar(bvm, wid)
        e1 = _bounds_scalar(bvm, wid + 1)
        ck0 = lax.div(e0, CHUNK)
        ck1 = lax.div(e1 + (CHUNK - 1), CHUNK)
        nck2 = 2 * lax.div(ck1 - ck0 + 1, 2)

        sidx = (sidx0, sidx1)
        didx = (didx0, didx1)
        sems = (sem0, sem1)

        def start(ck, b):
            pltpu.sync_copy(dst_hbm.at[pl.ds(ck * CHUNK, CHUNK)], didx[b])
            if with_gather:
                pltpu.sync_copy(src_hbm.at[pl.ds(ck * CHUNK, CHUNK)], sidx[b])
                pltpu.async_copy(g_hbm.at[sidx[b]], rows.at[b], sems[b])

        def process(ck, b):
            if with_gather:
                pltpu.make_async_copy(g_hbm.at[sidx[b]], rows.at[b],
                                      sems[b]).wait()
            dv0 = didx[b][pl.ds(0, 16)] - base
            cur = jnp.clip(dv0[0], 0, RPT - 1)
            avs = [acc[cur, pl.ds(k * 16, 16)] for k in range(nvec)]
            for g in range(CHUNK // 16):
                dv = didx[b][pl.ds(g * 16, 16)] - base
                for lane in range(16):
                    r = g * 16 + lane
                    glob = ck * CHUNK + r
                    valid = (glob >= e0) & (glob < e1)
                    d = dv[lane]
                    new = valid & (d != cur)
                    vf = jnp.where(valid, 1.0, 0.0).astype(jnp.float32)

                    avs_now = avs
                    cur_now = cur

                    @pl.when(new)
                    def _():
                        for k in range(nvec):
                            acc[cur_now, pl.ds(k * 16, 16)] = avs_now[k]

                    navs = []
                    for k in range(nvec):
                        if with_gather:
                            t = rows[b, r, pl.ds(k * 16, 16)] * vf
                        else:
                            t = jnp.zeros((16,), jnp.float32) + vf
                        navs.append(jnp.where(new, t, avs[k] + t))
                    avs = navs
                    cur = jnp.where(new, d, cur)
            for k in range(nvec):
                acc[cur, pl.ds(k * 16, 16)] = avs[k]

        start(ck0, 0)

        @pl.loop(ck0, ck0 + nck2, step=2)
        def _(ck):
            for b in range(2):
                start(ck + b + 1, 1 - b)
                process(ck + b, b)

        if with_gather:
            pltpu.make_async_copy(g_hbm.at[sidx[0]], rows.at[0],
                                  sems[0]).wait()
        pltpu.sync_copy(acc, out_hbm.at[pl.ds(base, RPT)])

    kern = pl.kernel(
        body,
        out_type=jax.ShapeDtypeStruct((N_PAD, width), jnp.float32),
        mesh=_mesh,
        scratch_types=[
            pltpu.VMEM((RPT, width), jnp.float32),
            pltpu.VMEM((CHUNK,), jnp.int32),
            pltpu.VMEM((CHUNK,), jnp.int32),
            pltpu.VMEM((CHUNK,), jnp.int32),
            pltpu.VMEM((CHUNK,), jnp.int32),
            pltpu.VMEM((NW + 16,), jnp.int32),
            pltpu.VMEM((2, CHUNK, width), jnp.float32),
            pltpu.SemaphoreType.DMA,
            pltpu.SemaphoreType.DMA,
        ],
    )
    return kern


_sc_deg = _make_sc_agg(16, with_gather=False)
_sc_agg128 = _make_sc_agg(128, with_gather=True)

BR = 256
GRID = N_PAD // BR


def _t1_body(deg_ref, f_ref, nrm_ref, g0_ref):
    deg = deg_ref[:, 0:1]
    nrm1 = jnp.where(deg > 0, lax.rsqrt(deg), 0.0)
    nrm = jnp.broadcast_to(nrm1, (BR, D))
    nrm_ref[...] = nrm
    g0_ref[...] = f_ref[...] * nrm


def _t2_body(a_ref, n_ref, w_ref, b_ref, out_ref):
    n = n_ref[...]
    x = a_ref[...] * n
    y = jnp.dot(x, w_ref[...], preferred_element_type=jnp.float32) + b_ref[...]
    out_ref[...] = jnp.maximum(y, 0.0) * n


def _t3_body(a_ref, n_ref, w1_ref, b1_ref, w2_ref, out_ref):
    n = n_ref[...]
    x = a_ref[...] * n
    h = jnp.dot(x, w1_ref[...], preferred_element_type=jnp.float32) + b1_ref[...]
    h = jnp.maximum(h, 0.0) * n
    out_ref[...] = jnp.dot(h, w2_ref[...], preferred_element_type=jnp.float32)


BR4 = 400
GRID4 = N // BR4


def _t4_body(a_ref, n_ref, b_ref, out_ref):
    out_ref[...] = a_ref[...] * n_ref[...] + b_ref[...]


def _row_spec(br, w):
    return pl.BlockSpec((br, w), lambda i: (i, 0))


def _full_spec(shape):
    return pl.BlockSpec(shape, lambda i: tuple(0 for _ in shape))


_t1 = pl.pallas_call(
    _t1_body,
    grid=(GRID,),
    in_specs=[_row_spec(BR, 16), _row_spec(BR, D)],
    out_specs=[_row_spec(BR, D), _row_spec(BR, D)],
    out_shape=[jax.ShapeDtypeStruct((N_PAD, D), jnp.float32),
               jax.ShapeDtypeStruct((N_PAD, D), jnp.float32)],
)

_t2 = pl.pallas_call(
    _t2_body,
    grid=(GRID,),
    in_specs=[_row_spec(BR, D), _row_spec(BR, D),
              _full_spec((D, D)), _full_spec((1, D))],
    out_specs=_row_spec(BR, D),
    out_shape=jax.ShapeDtypeStruct((N_PAD, D), jnp.float32),
)

_t3 = pl.pallas_call(
    _t3_body,
    grid=(GRID,),
    in_specs=[_row_spec(BR, D), _row_spec(BR, D),
              _full_spec((D, D)), _full_spec((1, D)), _full_spec((D, D))],
    out_specs=_row_spec(BR, D),
    out_shape=jax.ShapeDtypeStruct((N_PAD, D), jnp.float32),
)

_t4 = pl.pallas_call(
    _t4_body,
    grid=(GRID4,),
    in_specs=[_row_spec(BR4, D), _row_spec(BR4, D), _full_spec((1, D))],
    out_specs=_row_spec(BR4, D),
    out_shape=jax.ShapeDtypeStruct((N, D), jnp.float32),
)


def kernel(features, edge_index, W0, b0, W1, b1, W2, b2):
    f32 = jnp.float32
    ei = edge_index.astype(jnp.int32)
    order = jnp.argsort(ei[1])
    dst_u = ei[1][order]
    bounds = jnp.searchsorted(dst_u, jnp.arange(NW + 1, dtype=jnp.int32) * RPT
                              ).astype(jnp.int32)
    src_s = jnp.concatenate([ei[0][order], jnp.zeros((3 * CHUNK,), jnp.int32)])
    dst_s = jnp.concatenate([dst_u,
                             jnp.full((3 * CHUNK,), N_PAD - 1, jnp.int32)])
    bounds_p = jnp.zeros((NW + 16,), jnp.int32).at[:NW + 1].set(bounds)

    feat_p = jnp.zeros((N_PAD, D), f32).at[:N].set(features.astype(f32))
    w2p = jnp.zeros((D, D), f32).at[:, :C].set(W2.astype(f32))
    b2p = jnp.zeros((1, D), f32).at[0, :C].set(b2.astype(f32))
    z16 = jnp.zeros((RPT, 16), f32)
    z128 = jnp.zeros((RPT, D), f32)

    deg = _sc_deg(z16, src_s, dst_s, bounds_p, z16)
    nrm, g0 = _t1(deg, feat_p)
    a0 = _sc_agg128(g0, src_s, dst_s, bounds_p, z128)
    g1 = _t2(a0, nrm, W0.astype(f32), b0.reshape(1, D).astype(f32))
    a1 = _sc_agg128(g1, src_s, dst_s, bounds_p, z128)
    t2 = _t3(a1, nrm, W1.astype(f32), b1.reshape(1, D).astype(f32), w2p)
    a2 = _sc_agg128(t2, src_s, dst_s, bounds_p, z128)
    out = _t4(a2, nrm, b2p)
    return out[:, :C]


# speedup vs baseline: 1.0833x; 1.0833x over previous
import functools

import jax
import jax.numpy as jnp
from jax import lax
from jax.experimental import pallas as pl
from jax.experimental.pallas import tpu as pltpu
from jax.experimental.pallas import tpu_sc as plsc

N = 10000
D = 128
C = 40
E = 320000

NC = 2
NS = 16
NW = NC * NS
N_PAD = 10240
RPT = N_PAD // NW
CHUNK = 32

_mesh = plsc.VectorSubcoreMesh(
    core_axis_name="c", subcore_axis_name="s", num_cores=NC, num_subcores=NS)


def _bounds_scalar(bv, t):
    total = jnp.zeros((), jnp.int32)
    for q in range(3):
        v = bv[pl.ds(q * 16, 16)]
        for lane in range(16):
            i = q * 16 + lane
            if i <= NW:
                total = jnp.where(t == i, v[lane], total)
    return total


def _make_sc_agg(width, with_gather):
    nvec = width // 16

    def body(g_hbm, src_hbm, dst_hbm, bounds_hbm, zeros_hbm, out_hbm,
             acc, sidx0, sidx1, didx0, didx1, bvm, rows, sem0, sem1):
        cid = lax.axis_index("c")
        sid = lax.axis_index("s")
        wid = cid * NS + sid
        base = wid * RPT

        pltpu.sync_copy(zeros_hbm, acc)
        pltpu.sync_copy(bounds_hbm, bvm)
        e0 = _bounds_scalar(bvm, wid)
        e1 = _bounds_scalar(bvm, wid + 1)
        ck0 = lax.div(e0, CHUNK)
        ck1 = lax.div(e1 + (CHUNK - 1), CHUNK)
        nck2 = 2 * lax.div(ck1 - ck0 + 1, 2)

        sidx = (sidx0, sidx1)
        didx = (didx0, didx1)
        sems = (sem0, sem1)

        def start(ck, b):
            pltpu.sync_copy(dst_hbm.at[pl.ds(ck * CHUNK, CHUNK)], didx[b])
            if with_gather:
                pltpu.sync_copy(src_hbm.at[pl.ds(ck * CHUNK, CHUNK)], sidx[b])
                pltpu.async_copy(g_hbm.at[sidx[b]], rows.at[b], sems[b])

        def process(ck, b):
            if with_gather:
                pltpu.make_async_copy(g_hbm.at[sidx[b]], rows.at[b],
                                      sems[b]).wait()
            dv0 = didx[b][pl.ds(0, 16)] - base
            cur = jnp.clip(dv0[0], 0, RPT - 1)
            avs = [acc[cur, pl.ds(k * 16, 16)] for k in range(nvec)]
            for g in range(CHUNK // 16):
                dv = didx[b][pl.ds(g * 16, 16)] - base
                for lane in range(16):
                    r = g * 16 + lane
                    glob = ck * CHUNK + r
                    valid = (glob >= e0) & (glob < e1)
                    d = dv[lane]
                    new = valid & (d != cur)
                    vf = jnp.where(valid, 1.0, 0.0).astype(jnp.float32)

                    avs_now = avs
                    cur_now = cur

                    @pl.when(new)
                    def _():
                        for k in range(nvec):
                            acc[cur_now, pl.ds(k * 16, 16)] = avs_now[k]

                    navs = []
                    for k in range(nvec):
                        if with_gather:
                            t = rows[b, r, pl.ds(k * 16, 16)] * vf
                        else:
                            t = jnp.zeros((16,), jnp.float32) + vf
                        navs.append(jnp.where(new, t, avs[k] + t))
                    avs = navs
                    cur = jnp.where(new, d, cur)
            for k in range(nvec):
                acc[cur, pl.ds(k * 16, 16)] = avs[k]

        start(ck0, 0)

        @pl.loop(ck0, ck0 + nck2, step=2)
        def _(ck):
            for b in range(2):
                start(ck + b + 1, 1 - b)
                process(ck + b, b)

        if with_gather:
            pltpu.make_async_copy(g_hbm.at[sidx[0]], rows.at[0],
                                  sems[0]).wait()
        pltpu.sync_copy(acc, out_hbm.at[pl.ds(base, RPT)])

    kern = pl.kernel(
        body,
        out_type=jax.ShapeDtypeStruct((N_PAD, width), jnp.float32),
        mesh=_mesh,
        scratch_types=[
            pltpu.VMEM((RPT, width), jnp.float32),
            pltpu.VMEM((CHUNK,), jnp.int32),
            pltpu.VMEM((CHUNK,), jnp.int32),
            pltpu.VMEM((CHUNK,), jnp.int32),
            pltpu.VMEM((CHUNK,), jnp.int32),
            pltpu.VMEM((NW + 16,), jnp.int32),
            pltpu.VMEM((2, CHUNK, width), jnp.float32),
            pltpu.SemaphoreType.DMA,
            pltpu.SemaphoreType.DMA,
        ],
    )
    return kern


_sc_deg = _make_sc_agg(16, with_gather=False)
_sc_agg128 = _make_sc_agg(128, with_gather=True)

BR = 256
GRID = N_PAD // BR


def _t1_body(deg_ref, f_ref, nrm_ref, g0_ref):
    deg = deg_ref[:, 0:1]
    nrm1 = jnp.where(deg > 0, lax.rsqrt(deg), 0.0)
    nrm = jnp.broadcast_to(nrm1, (BR, D))
    nrm_ref[...] = nrm
    g0_ref[...] = f_ref[...] * nrm


def _t2_body(a_ref, n_ref, w_ref, b_ref, out_ref):
    n = n_ref[...]
    x = a_ref[...] * n
    y = jnp.dot(x, w_ref[...], preferred_element_type=jnp.float32) + b_ref[...]
    out_ref[...] = jnp.maximum(y, 0.0) * n


def _t3_body(a_ref, n_ref, w1_ref, b1_ref, w2_ref, out_ref):
    n = n_ref[...]
    x = a_ref[...] * n
    h = jnp.dot(x, w1_ref[...], preferred_element_type=jnp.float32) + b1_ref[...]
    h = jnp.maximum(h, 0.0) * n
    out_ref[...] = jnp.dot(h, w2_ref[...], preferred_element_type=jnp.float32)


BR4 = 400
GRID4 = N // BR4


def _t4_body(a_ref, n_ref, b_ref, out_ref):
    out_ref[...] = a_ref[...] * n_ref[...] + b_ref[...]


def _row_spec(br, w):
    return pl.BlockSpec((br, w), lambda i: (i, 0))


def _full_spec(shape):
    return pl.BlockSpec(shape, lambda i: tuple(0 for _ in shape))


_t1 = pl.pallas_call(
    _t1_body,
    grid=(GRID,),
    in_specs=[_row_spec(BR, 16), _row_spec(BR, D)],
    out_specs=[_row_spec(BR, D), _row_spec(BR, D)],
    out_shape=[jax.ShapeDtypeStruct((N_PAD, D), jnp.float32),
               jax.ShapeDtypeStruct((N_PAD, D), jnp.float32)],
)

_t2 = pl.pallas_call(
    _t2_body,
    grid=(GRID,),
    in_specs=[_row_spec(BR, D), _row_spec(BR, D),
              _full_spec((D, D)), _full_spec((1, D))],
    out_specs=_row_spec(BR, D),
    out_shape=jax.ShapeDtypeStruct((N_PAD, D), jnp.float32),
)

_t3 = pl.pallas_call(
    _t3_body,
    grid=(GRID,),
    in_specs=[_row_spec(BR, D), _row_spec(BR, D),
              _full_spec((D, D)), _full_spec((1, D)), _full_spec((D, D))],
    out_specs=_row_spec(BR, D),
    out_shape=jax.ShapeDtypeStruct((N_PAD, D), jnp.float32),
)

_t4 = pl.pallas_call(
    _t4_body,
    grid=(GRID4,),
    in_specs=[_row_spec(BR4, D), _row_spec(BR4, D), _full_spec((1, D))],
    out_specs=_row_spec(BR4, D),
    out_shape=jax.ShapeDtypeStruct((N, D), jnp.float32),
)


def kernel(features, edge_index, W0, b0, W1, b1, W2, b2):
    f32 = jnp.float32
    ei = edge_index.astype(jnp.int32)
    key = jnp.sort(ei[1] * 16384 + ei[0])
    dst_u = key >> 14
    src_u = key & 16383
    bounds = jnp.searchsorted(dst_u, jnp.arange(NW + 1, dtype=jnp.int32) * RPT
                              ).astype(jnp.int32)
    src_s = jnp.concatenate([src_u, jnp.zeros((3 * CHUNK,), jnp.int32)])
    dst_s = jnp.concatenate([dst_u,
                             jnp.full((3 * CHUNK,), N_PAD - 1, jnp.int32)])
    bounds_p = jnp.zeros((NW + 16,), jnp.int32).at[:NW + 1].set(bounds)

    feat_p = jnp.zeros((N_PAD, D), f32).at[:N].set(features.astype(f32))
    w2p = jnp.zeros((D, D), f32).at[:, :C].set(W2.astype(f32))
    b2p = jnp.zeros((1, D), f32).at[0, :C].set(b2.astype(f32))
    z16 = jnp.zeros((RPT, 16), f32)
    z128 = jnp.zeros((RPT, D), f32)

    deg = _sc_deg(z16, src_s, dst_s, bounds_p, z16)
    nrm, g0 = _t1(deg, feat_p)
    a0 = _sc_agg128(g0, src_s, dst_s, bounds_p, z128)
    g1 = _t2(a0, nrm, W0.astype(f32), b0.reshape(1, D).astype(f32))
    a1 = _sc_agg128(g1, src_s, dst_s, bounds_p, z128)
    t2 = _t3(a1, nrm, W1.astype(f32), b1.reshape(1, D).astype(f32), w2p)
    a2 = _sc_agg128(t2, src_s, dst_s, bounds_p, z128)
    out = _t4(a2, nrm, b2p)
    return out[:, :C]


# speedup vs baseline: 1.1452x; 1.0571x over previous
import functools

import jax
import jax.numpy as jnp
from jax import lax
from jax.experimental import pallas as pl
from jax.experimental.pallas import tpu as pltpu
from jax.experimental.pallas import tpu_sc as plsc

N = 10000
D = 128
C = 40
E = 320000

NC = 2
NS = 16
NW = NC * NS
N_PAD = 10240
RPT = N_PAD // NW
CHUNK = 32

_mesh = plsc.VectorSubcoreMesh(
    core_axis_name="c", subcore_axis_name="s", num_cores=NC, num_subcores=NS)


def _bounds_scalar(bv, t):
    total = jnp.zeros((), jnp.int32)
    for q in range(3):
        v = bv[pl.ds(q * 16, 16)]
        for lane in range(16):
            i = q * 16 + lane
            if i <= NW:
                total = jnp.where(t == i, v[lane], total)
    return total


def _make_sc_agg(width, with_gather):
    nvec = width // 16

    def body(g_hbm, src_hbm, dst_hbm, bounds_hbm, zeros_hbm, out_hbm,
             acc, sidx0, sidx1, didx0, didx1, bvm, rows, sem0, sem1):
        cid = lax.axis_index("c")
        sid = lax.axis_index("s")
        wid = cid * NS + sid
        base = wid * RPT

        pltpu.sync_copy(zeros_hbm, acc)
        pltpu.sync_copy(bounds_hbm, bvm)
        e0 = _bounds_scalar(bvm, wid)
        e1 = _bounds_scalar(bvm, wid + 1)
        ck0 = lax.div(e0, CHUNK)
        ck1 = lax.div(e1 + (CHUNK - 1), CHUNK)
        nck2 = 2 * lax.div(ck1 - ck0 + 1, 2)

        sidx = (sidx0, sidx1)
        didx = (didx0, didx1)
        sems = (sem0, sem1)

        def start(ck, b):
            pltpu.sync_copy(dst_hbm.at[pl.ds(ck * CHUNK, CHUNK)], didx[b])
            if with_gather:
                pltpu.sync_copy(src_hbm.at[pl.ds(ck * CHUNK, CHUNK)], sidx[b])
                pltpu.async_copy(g_hbm.at[sidx[b]], rows.at[b], sems[b])

        def wait_g(b):
            if with_gather:
                pltpu.make_async_copy(g_hbm.at[sidx[b]], rows.at[b],
                                      sems[b]).wait()

        def process(ck, b):
            wait_g(b)
            dv0 = didx[b][pl.ds(0, 16)] - base
            cur = jnp.clip(dv0[0], 0, RPT - 1)
            avs = [acc[cur, pl.ds(k * 16, 16)] for k in range(nvec)]
            for g in range(CHUNK // 16):
                dv = didx[b][pl.ds(g * 16, 16)] - base
                for lane in range(16):
                    r = g * 16 + lane
                    d = dv[lane]
                    new = d != cur
                    avs_now = avs
                    cur_now = cur

                    @pl.when(new)
                    def _():
                        for k in range(nvec):
                            acc[cur_now, pl.ds(k * 16, 16)] = avs_now[k]

                    navs = []
                    for k in range(nvec):
                        if with_gather:
                            t = rows[b, r, pl.ds(k * 16, 16)]
                        else:
                            t = jnp.ones((16,), jnp.float32)
                        navs.append(jnp.where(new, t, avs[k] + t))
                    avs = navs
                    cur = jnp.where(new, d, cur)
            for k in range(nvec):
                acc[cur, pl.ds(k * 16, 16)] = avs[k]

        def process_masked(ck, b):
            wait_g(b)
            for g in range(CHUNK // 16):
                dv = didx[b][pl.ds(g * 16, 16)] - base
                for lane in range(16):
                    r = g * 16 + lane
                    glob = ck * CHUNK + r
                    valid = (glob >= e0) & (glob < e1)
                    d = dv[lane]

                    @pl.when(valid)
                    def _():
                        for k in range(nvec):
                            sl = pl.ds(k * 16, 16)
                            if with_gather:
                                acc[d, sl] = acc[d, sl] + rows[b, r, sl]
                            else:
                                acc[d, sl] = acc[d, sl] + 1.0

        start(ck0, 0)
        process_masked(ck0, 0)

        ilo = ck0 + 1
        n = ck1 - ilo
        m2e = 2 * lax.div(jnp.maximum(n - 1, 0), 2)
        start(ilo, 0)

        @pl.loop(ilo, ilo + m2e, step=2)
        def _(ck):
            for b in range(2):
                start(ck + b + 1, 1 - b)
                process(ck + b, b)

        rem = ilo + m2e

        @pl.when(rem < ck1)
        def _():
            process_masked(rem, 0)

            @pl.when(rem + 1 < ck1)
            def _():
                start(rem + 1, 1)
                process_masked(rem + 1, 1)

        if with_gather:
            @pl.when(rem >= ck1)
            def _():
                wait_g(0)

        pltpu.sync_copy(acc, out_hbm.at[pl.ds(base, RPT)])

    kern = pl.kernel(
        body,
        out_type=jax.ShapeDtypeStruct((N_PAD, width), jnp.float32),
        mesh=_mesh,
        scratch_types=[
            pltpu.VMEM((RPT, width), jnp.float32),
            pltpu.VMEM((CHUNK,), jnp.int32),
            pltpu.VMEM((CHUNK,), jnp.int32),
            pltpu.VMEM((CHUNK,), jnp.int32),
            pltpu.VMEM((CHUNK,), jnp.int32),
            pltpu.VMEM((NW + 16,), jnp.int32),
            pltpu.VMEM((2, CHUNK, width), jnp.float32),
            pltpu.SemaphoreType.DMA,
            pltpu.SemaphoreType.DMA,
        ],
    )
    return kern


_sc_deg = _make_sc_agg(16, with_gather=False)
_sc_agg128 = _make_sc_agg(128, with_gather=True)

BR = 256
GRID = N_PAD // BR


def _t1_body(deg_ref, f_ref, nrm_ref, g0_ref):
    deg = deg_ref[:, 0:1]
    nrm1 = jnp.where(deg > 0, lax.rsqrt(deg), 0.0)
    nrm = jnp.broadcast_to(nrm1, (BR, D))
    nrm_ref[...] = nrm
    g0_ref[...] = f_ref[...] * nrm


def _t2_body(a_ref, n_ref, w_ref, b_ref, out_ref):
    n = n_ref[...]
    x = a_ref[...] * n
    y = jnp.dot(x, w_ref[...], preferred_element_type=jnp.float32) + b_ref[...]
    out_ref[...] = jnp.maximum(y, 0.0) * n


def _t3_body(a_ref, n_ref, w1_ref, b1_ref, w2_ref, out_ref):
    n = n_ref[...]
    x = a_ref[...] * n
    h = jnp.dot(x, w1_ref[...], preferred_element_type=jnp.float32) + b1_ref[...]
    h = jnp.maximum(h, 0.0) * n
    out_ref[...] = jnp.dot(h, w2_ref[...], preferred_element_type=jnp.float32)


BR4 = 400
GRID4 = N // BR4


def _t4_body(a_ref, n_ref, b_ref, out_ref):
    out_ref[...] = a_ref[...] * n_ref[...] + b_ref[...]


def _row_spec(br, w):
    return pl.BlockSpec((br, w), lambda i: (i, 0))


def _full_spec(shape):
    return pl.BlockSpec(shape, lambda i: tuple(0 for _ in shape))


_t1 = pl.pallas_call(
    _t1_body,
    grid=(GRID,),
    in_specs=[_row_spec(BR, 16), _row_spec(BR, D)],
    out_specs=[_row_spec(BR, D), _row_spec(BR, D)],
    out_shape=[jax.ShapeDtypeStruct((N_PAD, D), jnp.float32),
               jax.ShapeDtypeStruct((N_PAD, D), jnp.float32)],
)

_t2 = pl.pallas_call(
    _t2_body,
    grid=(GRID,),
    in_specs=[_row_spec(BR, D), _row_spec(BR, D),
              _full_spec((D, D)), _full_spec((1, D))],
    out_specs=_row_spec(BR, D),
    out_shape=jax.ShapeDtypeStruct((N_PAD, D), jnp.float32),
)

_t3 = pl.pallas_call(
    _t3_body,
    grid=(GRID,),
    in_specs=[_row_spec(BR, D), _row_spec(BR, D),
              _full_spec((D, D)), _full_spec((1, D)), _full_spec((D, D))],
    out_specs=_row_spec(BR, D),
    out_shape=jax.ShapeDtypeStruct((N_PAD, D), jnp.float32),
)

_t4 = pl.pallas_call(
    _t4_body,
    grid=(GRID4,),
    in_specs=[_row_spec(BR4, D), _row_spec(BR4, D), _full_spec((1, D))],
    out_specs=_row_spec(BR4, D),
    out_shape=jax.ShapeDtypeStruct((N, D), jnp.float32),
)


def kernel(features, edge_index, W0, b0, W1, b1, W2, b2):
    f32 = jnp.float32
    ei = edge_index.astype(jnp.int32)
    key = jnp.sort(ei[1] * 16384 + ei[0])
    dst_u = key >> 14
    src_u = key & 16383
    bounds = jnp.searchsorted(dst_u, jnp.arange(NW + 1, dtype=jnp.int32) * RPT
                              ).astype(jnp.int32)
    src_s = jnp.concatenate([src_u, jnp.zeros((3 * CHUNK,), jnp.int32)])
    dst_s = jnp.concatenate([dst_u,
                             jnp.full((3 * CHUNK,), N_PAD - 1, jnp.int32)])
    bounds_p = jnp.zeros((NW + 16,), jnp.int32).at[:NW + 1].set(bounds)

    feat_p = jnp.zeros((N_PAD, D), f32).at[:N].set(features.astype(f32))
    w2p = jnp.zeros((D, D), f32).at[:, :C].set(W2.astype(f32))
    b2p = jnp.zeros((1, D), f32).at[0, :C].set(b2.astype(f32))
    z16 = jnp.zeros((RPT, 16), f32)
    z128 = jnp.zeros((RPT, D), f32)

    deg = _sc_deg(z16, src_s, dst_s, bounds_p, z16)
    nrm, g0 = _t1(deg, feat_p)
    a0 = _sc_agg128(g0, src_s, dst_s, bounds_p, z128)
    g1 = _t2(a0, nrm, W0.astype(f32), b0.reshape(1, D).astype(f32))
    a1 = _sc_agg128(g1, src_s, dst_s, bounds_p, z128)
    t2 = _t3(a1, nrm, W1.astype(f32), b1.reshape(1, D).astype(f32), w2p)
    a2 = _sc_agg128(t2, src_s, dst_s, bounds_p, z128)
    out = _t4(a2, nrm, b2p)
    return out[:, :C]
